# baseline (device time: 14489 ns/iter reference)
import jax
import jax.numpy as jnp
from jax import lax
from jax.experimental import pallas as pl
from jax.experimental.pallas import tpu as pltpu

N_DEV = 4
B, SQ, SKV, DH = 2, 128, 128, 64
H_LOC = 4
D_MODEL = 512
D_LOC = H_LOC * DH
N_COL = 4
D_CHUNK = D_MODEL // N_COL
N_CHUNK = B * N_COL


def kernel(x, Wq, K_ext, V_ext, Wo):
    def body(x_ref, wq_ref, k_ref, v_ref, wo_ref, out_ref,
             kh_ref, vh_ref, kv_sems,
             send_ref, recv_ref, send_sems, recv_sems):
        my_pos = lax.axis_index("i")
        heads = pl.ds(lax.rem(my_pos, 2) * H_LOC, H_LOC)
        k_copy = pltpu.make_async_copy(
            k_ref.at[:, heads, :], kh_ref, kv_sems.at[0]
        )
        v_copy = pltpu.make_async_copy(
            v_ref.at[:, heads, :], vh_ref, kv_sems.at[1]
        )
        k_copy.start()
        v_copy.start()
        pa = my_pos ^ 1
        pb = (N_DEV - 1) - my_pos

        def partners(c):
            return (pa, pb) if c % 2 == 0 else (pb, pa)

        barrier_sem = pltpu.get_barrier_semaphore()
        for nbr in (pa, pb):
            pl.semaphore_signal(
                barrier_sem, inc=1,
                device_id=(nbr,), device_id_type=pl.DeviceIdType.MESH,
            )

        wq = wq_ref[:].astype(jnp.bfloat16)
        wo = wo_ref[:].astype(jnp.bfloat16)

        def exchange(round_idx, c, partner):
            return pltpu.make_async_remote_copy(
                src_ref=send_ref.at[round_idx, c],
                dst_ref=recv_ref.at[round_idx, c],
                send_sem=send_sems.at[round_idx, c],
                recv_sem=recv_sems.at[round_idx, c],
                device_id=(partner,),
                device_id_type=pl.DeviceIdType.MESH,
            )

        r1 = []
        for b in range(B):
            xb = x_ref[b].astype(jnp.bfloat16)
            q_b = lax.dot(xb, wq, preferred_element_type=jnp.float32)
            q_b = (q_b * 0.125).astype(jnp.bfloat16)
            if b == 0:
                k_copy.wait()
                v_copy.wait()
            ctx = []
            for h in range(H_LOC):
                q_bh = q_b[:, h * DH:(h + 1) * DH]
                rows = slice(b * SKV, (b + 1) * SKV)
                k_bh = kh_ref[rows, h, :].astype(jnp.bfloat16)
                v_bh = vh_ref[rows, h, :].astype(jnp.bfloat16)
                s = lax.dot_general(
                    q_bh, k_bh, (((1,), (1,)), ((), ())),
                    preferred_element_type=jnp.float32,
                )
                e = jnp.exp(s)
                recip = 1.0 / jnp.sum(e, axis=1, keepdims=True)
                ctx_bh = lax.dot(
                    e.astype(jnp.bfloat16), v_bh,
                    preferred_element_type=jnp.float32,
                ) * recip
                ctx.append(ctx_bh.astype(jnp.bfloat16))

            ctx_b = jnp.concatenate(ctx, axis=1)
            for j in range(N_COL):
                c = b * N_COL + j
                cols = slice(j * D_CHUNK, (j + 1) * D_CHUNK)
                partial = lax.dot(
                    ctx_b, wo[:, cols], preferred_element_type=jnp.float32
                )
                send_ref[0, c] = partial.astype(jnp.bfloat16)
                if c == 0:
                    pl.semaphore_wait(barrier_sem, 2)
                rdma = exchange(0, c, partners(c)[0])
                rdma.start()
                r1.append(rdma)

        r2 = []
        for c in range(N_CHUNK):
            r1[c].wait_recv()
            send_ref[1, c] = send_ref[0, c] + recv_ref[0, c]
            rdma = exchange(1, c, partners(c)[1])
            rdma.start()
            r2.append(rdma)

        for c in range(N_CHUNK):
            b, j = divmod(c, N_COL)
            cols = slice(j * D_CHUNK, (j + 1) * D_CHUNK)
            r2[c].wait_recv()
            out_ref[b, :, cols] = send_ref[1, c] + recv_ref[1, c]

        for rdma in r1 + r2:
            rdma.wait_send()

    def kv_map(i):
        return (0, lax.axis_index("i") // 2, 0)

    return pl.pallas_call(
        body,
        grid=(1,),
        out_shape=jax.ShapeDtypeStruct((B, SQ, D_MODEL), jnp.bfloat16),
        in_specs=[
            pl.BlockSpec(memory_space=pltpu.VMEM),
            pl.BlockSpec(memory_space=pltpu.VMEM),
            pl.BlockSpec((B * SKV, 8, DH), kv_map),
            pl.BlockSpec((B * SKV, 8, DH), kv_map),
            pl.BlockSpec(memory_space=pltpu.VMEM),
        ],
        out_specs=pl.BlockSpec((B, SQ, D_MODEL), lambda i: (0, 0, 0)),
        scratch_shapes=[
            pltpu.VMEM((B * SKV, H_LOC, DH), jnp.float32),
            pltpu.VMEM((B * SKV, H_LOC, DH), jnp.float32),
            pltpu.SemaphoreType.DMA((2,)),
            pltpu.VMEM((2, N_CHUNK, SQ, D_CHUNK), jnp.bfloat16),
            pltpu.VMEM((2, N_CHUNK, SQ, D_CHUNK), jnp.bfloat16),
            pltpu.SemaphoreType.DMA((2, N_CHUNK)),
            pltpu.SemaphoreType.DMA((2, N_CHUNK)),
        ],
        compiler_params=pltpu.CompilerParams(collective_id=0),
    )(
        x,
        Wq,
        K_ext.reshape(B * SKV, 16, DH),
        V_ext.reshape(B * SKV, 16, DH),
        Wo,
    )


# device time: 13269 ns/iter; 1.0919x vs baseline; 1.0919x over previous
import jax
import jax.numpy as jnp
from jax import lax
from jax.experimental import pallas as pl
from jax.experimental.pallas import tpu as pltpu

N_DEV = 4
B, SQ, SKV, DH = 2, 128, 128, 64
H_LOC = 4
D_MODEL = 512
D_LOC = H_LOC * DH
N_COL = 4
D_CHUNK = D_MODEL // N_COL
N_CHUNK = B * N_COL


def kernel(x, Wq, K_ext, V_ext, Wo):
    def body(x_ref, wq_ref, k_ref, v_ref, wo_ref, out_ref,
             send_ref, recv_ref, send_sems, recv_sems):
        my_pos = lax.axis_index("i")
        pa = my_pos ^ 1
        pb = (N_DEV - 1) - my_pos

        def partners(c):
            return (pa, pb) if c % 2 == 0 else (pb, pa)

        barrier_sem = pltpu.get_barrier_semaphore()
        for nbr in (pa, pb):
            pl.semaphore_signal(
                barrier_sem, inc=1,
                device_id=(nbr,), device_id_type=pl.DeviceIdType.MESH,
            )

        wq = wq_ref[:].astype(jnp.bfloat16)
        wo = wo_ref[:].astype(jnp.bfloat16)

        def exchange(round_idx, c, partner):
            return pltpu.make_async_remote_copy(
                src_ref=send_ref.at[round_idx, c],
                dst_ref=recv_ref.at[round_idx, c],
                send_sem=send_sems.at[round_idx, c],
                recv_sem=recv_sems.at[round_idx, c],
                device_id=(partner,),
                device_id_type=pl.DeviceIdType.MESH,
            )

        r1 = []
        for b in range(B):
            xb = x_ref[b].astype(jnp.bfloat16)
            q_b = lax.dot(xb, wq, preferred_element_type=jnp.float32)
            q_b = (q_b * 0.125).astype(jnp.bfloat16)
            ctx = []
            for h in range(H_LOC):
                q_bh = q_b[:, h * DH:(h + 1) * DH]
                k_bh = k_ref[b, :, h * DH:(h + 1) * DH]
                v_bh = v_ref[b, :, h * DH:(h + 1) * DH]
                s = lax.dot_general(
                    q_bh, k_bh, (((1,), (1,)), ((), ())),
                    preferred_element_type=jnp.float32,
                )
                e = jnp.exp(s)
                recip = 1.0 / jnp.sum(e, axis=1, keepdims=True)
                ctx_bh = lax.dot(
                    e.astype(jnp.bfloat16), v_bh,
                    preferred_element_type=jnp.float32,
                ) * recip
                ctx.append(ctx_bh.astype(jnp.bfloat16))

            ctx_b = jnp.concatenate(ctx, axis=1)
            for j in range(N_COL):
                c = b * N_COL + j
                cols = slice(j * D_CHUNK, (j + 1) * D_CHUNK)
                partial = lax.dot(
                    ctx_b, wo[:, cols], preferred_element_type=jnp.float32
                )
                send_ref[0, c] = partial.astype(jnp.bfloat16)
                if c == 0:
                    pl.semaphore_wait(barrier_sem, 2)
                rdma = exchange(0, c, partners(c)[0])
                rdma.start()
                r1.append(rdma)

        r2 = []
        for c in range(N_CHUNK):
            r1[c].wait_recv()
            send_ref[1, c] = send_ref[0, c] + recv_ref[0, c]
            rdma = exchange(1, c, partners(c)[1])
            rdma.start()
            r2.append(rdma)

        for c in range(N_CHUNK):
            b, j = divmod(c, N_COL)
            cols = slice(j * D_CHUNK, (j + 1) * D_CHUNK)
            r2[c].wait_recv()
            out_ref[b, :, cols] = send_ref[1, c] + recv_ref[1, c]

        for rdma in r1 + r2:
            rdma.wait_send()

    def kv_map(i):
        return (0, 0, lax.axis_index("i"))

    return pl.pallas_call(
        body,
        grid=(1,),
        out_shape=jax.ShapeDtypeStruct((B, SQ, D_MODEL), jnp.bfloat16),
        in_specs=[
            pl.BlockSpec((B, SQ, D_MODEL), lambda i: (0, 0, 0)),
            pl.BlockSpec((D_MODEL, D_LOC), lambda i: (0, 0)),
            pl.BlockSpec((B, SKV, D_LOC), kv_map),
            pl.BlockSpec((B, SKV, D_LOC), kv_map),
            pl.BlockSpec((D_LOC, D_MODEL), lambda i: (0, 0)),
        ],
        out_specs=pl.BlockSpec((B, SQ, D_MODEL), lambda i: (0, 0, 0)),
        scratch_shapes=[
            pltpu.VMEM((2, N_CHUNK, SQ, D_CHUNK), jnp.bfloat16),
            pltpu.VMEM((2, N_CHUNK, SQ, D_CHUNK), jnp.bfloat16),
            pltpu.SemaphoreType.DMA((2, N_CHUNK)),
            pltpu.SemaphoreType.DMA((2, N_CHUNK)),
        ],
        compiler_params=pltpu.CompilerParams(collective_id=0),
    )(
        x,
        Wq,
        K_ext.astype(jnp.bfloat16).reshape(B, SKV, 16 * DH),
        V_ext.astype(jnp.bfloat16).reshape(B, SKV, 16 * DH),
        Wo,
    )
